# baseline (speedup 1.0000x reference)
import functools

import jax
import jax.numpy as jnp
from jax import lax
from jax.experimental import pallas as pl
from jax.experimental.pallas import tpu as pltpu
from jax.experimental.pallas import tpu_sc as plsc

N = 10000
E = 320000
D = 128

NC = 2
NS = 16
NW = NC * NS
NPW = 320
WIN = 800
NWIN = E // WIN
NV = WIN // 16
CHUNK = 16


def _tc_pool(x, Wp, bp):
    def body(x_ref, w_ref, b_ref, o_ref):
        acc = jnp.dot(x_ref[...], w_ref[...], preferred_element_type=jnp.float32)
        o_ref[...] = jnp.maximum(acc + b_ref[...], 0.0)
    return pl.pallas_call(
        body,
        out_shape=jax.ShapeDtypeStruct((N, D), jnp.float32),
    )(x, Wp, bp.reshape(1, D))


def _tc_out(h, agg, Ws, Wn, b):
    def body(h_ref, a_ref, ws_ref, wn_ref, b_ref, o_ref):
        r = jnp.dot(h_ref[...], ws_ref[...], preferred_element_type=jnp.float32)
        r += jnp.dot(a_ref[...], wn_ref[...], preferred_element_type=jnp.float32)
        r += b_ref[...]
        n = jnp.sqrt(jnp.sum(r * r, axis=1, keepdims=True))
        r = r / jnp.maximum(n, 1e-12)
        o_ref[...] = jnp.maximum(r, 0.0)
    return pl.pallas_call(
        body,
        out_shape=jax.ShapeDtypeStruct((N, D), jnp.float32),
    )(h, agg, Ws, Wn, b.reshape(1, D))


def _sc_segmax(feat, edge_index):
    mesh = plsc.VectorSubcoreMesh(
        core_axis_name="c", subcore_axis_name="s",
        num_cores=NC, num_subcores=NS)

    @functools.partial(
        pl.kernel,
        out_type=jax.ShapeDtypeStruct((N, D), jnp.float32),
        mesh=mesh,
        compiler_params=pltpu.CompilerParams(needs_layout_passes=False),
        scratch_types=[
            pltpu.VMEM((NPW + 1, D), jnp.float32),
            pltpu.VMEM((WIN,), jnp.int32),
            pltpu.VMEM((WIN,), jnp.int32),
            pltpu.VMEM((WIN,), jnp.int32),
            pltpu.VMEM((WIN,), jnp.int32),
            pltpu.VMEM((64,), jnp.int32),
            pltpu.VMEM((64,), jnp.int32),
            pltpu.VMEM((WIN + 16,), jnp.int32),
            pltpu.VMEM((CHUNK,), jnp.int32),
            pltpu.VMEM((CHUNK,), jnp.int32),
            pltpu.VMEM((2, CHUNK, D), jnp.float32),
            pltpu.VMEM_SHARED((N, D), jnp.float32),
            pltpu.SemaphoreType.DMA,
            pltpu.SemaphoreType.DMA,
            pltpu.SemaphoreType.DMA,
            pltpu.SemaphoreType.DMA,
        ],
    )
    def segmax(feat_hbm, src_hbm, dst_hbm, out_hbm,
               aggl, sbuf0, sbuf1, dbuf0, dbuf1, cbuf, obuf, mbuf,
               idx0, idx1, rows, feat_sh,
               sem_e0, sem_e1, sem_g0, sem_g1):
        wid = lax.axis_index("s") * NC + lax.axis_index("c")
        lo = wid * NPW
        hi = lo + NPW
        sem_e = [sem_e0, sem_e1]
        sem_g = [sem_g0, sem_g1]
        idxb = [idx0, idx1]
        lane = jnp.arange(16, dtype=jnp.int32)
        lane0 = lane == 0

        @pl.when(lax.axis_index("s") == 0)
        def _():
            pltpu.sync_copy(feat_hbm, feat_sh)
        plsc.subcore_barrier()

        def zinit(i, _):
            aggl[i // (D // 16), pl.ds((i % (D // 16)) * 16, 16)] = (
                jnp.zeros((16,), jnp.float32))
            return 0
        lax.fori_loop(0, NPW * D // 16, zinit, 0)
        for g in range(4):
            cbuf[pl.ds(g * 16, 16)] = jnp.zeros((16,), jnp.int32)

        sbufs = [sbuf0, sbuf1]
        dbufs = [dbuf0, dbuf1]

        def edge_copies(win, b, s):
            eb = win * WIN
            return (pltpu.make_async_copy(src_hbm.at[pl.ds(eb, WIN)],
                                          sbufs[b], s),
                    pltpu.make_async_copy(dst_hbm.at[pl.ds(eb, WIN)],
                                          dbufs[b], s))

        def gather_copy(b):
            return pltpu.make_async_copy(feat_sh.at[idxb[b]], rows.at[b],
                                         sem_g[b])

        def build_idx(ch, b):
            v = mbuf[pl.ds(ch * CHUNK, CHUNK)]
            idxb[b][...] = jnp.minimum(v & 0x3FFF, N - 1)

        for cp in edge_copies(0, 0, sem_e0):
            cp.start()

        def window(win, b):
            srcv = sbufs[b]
            dstv = dbufs[b]

            @pl.when(win + 1 < NWIN)
            def _():
                for cp in edge_copies(win + 1, 1 - b, sem_e[1 - b]):
                    cp.start()
            for cp in edge_copies(win, b, sem_e[b]):
                cp.wait()

            @plsc.parallel_loop(0, NV, unroll=4)
            def _(v):
                d = dstv[pl.ds(v * 16, 16)]
                m = (d >= lo) & (d < hi)
                cnt = plsc.all_reduce_population_count(m)
                plsc.store_scatter(cbuf, [jnp.full((16,), v, jnp.int32)],
                                   cnt, mask=lane0)

            total = jnp.int32(0)
            for g in range(4):
                c = cbuf[pl.ds(g * 16, 16)]
                ex = plsc.cumsum(c) - c + total
                obuf[pl.ds(g * 16, 16)] = ex
                total = (ex + c)[15]
            nmatch = total

            @plsc.parallel_loop(0, NV, unroll=4)
            def _(v):
                d = dstv[pl.ds(v * 16, 16)]
                s = srcv[pl.ds(v * 16, 16)]
                m = (d >= lo) & (d < hi)
                combo = ((d - lo) << 14) | s
                base = plsc.load_gather(obuf, [jnp.full((16,), v, jnp.int32)])
                pos = base + plsc.cumsum(m.astype(jnp.int32)) - 1
                plsc.store_scatter(mbuf, [pos], combo, mask=m)

            mbuf[pl.ds(nmatch, 16)] = jnp.full((16,), NPW << 14, jnp.int32)

            nch = (nmatch + (CHUNK - 1)) // CHUNK

            @pl.when(nch > 0)
            def _():
                build_idx(0, 0)
                gather_copy(0).start()

            def couter(o, _):
                for cb in range(2):
                    ch = o * 2 + cb

                    @pl.when(ch < nch)
                    def _():
                        gather_copy(cb).wait()

                        @pl.when(ch + 1 < nch)
                        def _():
                            build_idx(ch + 1, 1 - cb)
                            gather_copy(1 - cb).start()

                        rr = rows.at[cb]

                        def rmw(j, _):
                            combo = mbuf[pl.ds(ch * CHUNK + j, 16)][0]
                            dl = combo >> 14
                            for c in range(D // 16):
                                a = aggl[dl, pl.ds(c * 16, 16)]
                                r = rr[j, pl.ds(c * 16, 16)]
                                aggl[dl, pl.ds(c * 16, 16)] = jnp.maximum(a, r)
                            return 0
                        lax.fori_loop(0, CHUNK, rmw, 0, unroll=4)
                return 0
            lax.fori_loop(0, (nch + 1) // 2, couter, 0)
            return 0

        def wouter(o, _):
            for b in range(2):
                window(o * 2 + b, b)
            return 0
        lax.fori_loop(0, NWIN // 2, wouter, 0)

        @pl.when(wid < NW - 1)
        def _():
            pltpu.sync_copy(aggl.at[pl.ds(0, NPW)], out_hbm.at[pl.ds(lo, NPW)])

        @pl.when(wid == NW - 1)
        def _():
            pltpu.sync_copy(aggl.at[pl.ds(0, N - (NW - 1) * NPW)],
                            out_hbm.at[pl.ds((NW - 1) * NPW, N - (NW - 1) * NPW)])

    return segmax(feat, edge_index[0], edge_index[1])


def kernel(inputs, edge_index, Wp1, bp1, Ws1, Wn1, b1, Wp2, bp2, Ws2, Wn2, b2):
    feat1 = _tc_pool(inputs, Wp1, bp1)
    agg1 = _sc_segmax(feat1, edge_index)
    h1 = _tc_out(inputs, agg1, Ws1, Wn1, b1)
    feat2 = _tc_pool(h1, Wp2, bp2)
    agg2 = _sc_segmax(feat2, edge_index)
    h2 = _tc_out(h1, agg2, Ws2, Wn2, b2)
    return h2


# speedup vs baseline: 1.0085x; 1.0085x over previous
import functools

import jax
import jax.numpy as jnp
from jax import lax
from jax.experimental import pallas as pl
from jax.experimental.pallas import tpu as pltpu
from jax.experimental.pallas import tpu_sc as plsc

N = 10000
E = 320000
D = 128

NC = 2
NS = 16
NW = NC * NS
NPW = 320
WIN = 800
NWIN = E // WIN
NV = WIN // 16
CHUNK = 16


def _tc_pool(x, Wp, bp):
    def body(x_ref, w_ref, b_ref, o_ref):
        acc = jnp.dot(x_ref[...], w_ref[...], preferred_element_type=jnp.float32)
        o_ref[...] = jnp.maximum(acc + b_ref[...], 0.0)
    return pl.pallas_call(
        body,
        out_shape=jax.ShapeDtypeStruct((N, D), jnp.float32),
    )(x, Wp, bp.reshape(1, D))


def _tc_out(h, agg, Ws, Wn, b):
    def body(h_ref, a_ref, ws_ref, wn_ref, b_ref, o_ref):
        r = jnp.dot(h_ref[...], ws_ref[...], preferred_element_type=jnp.float32)
        r += jnp.dot(a_ref[...], wn_ref[...], preferred_element_type=jnp.float32)
        r += b_ref[...]
        n = jnp.sqrt(jnp.sum(r * r, axis=1, keepdims=True))
        r = r / jnp.maximum(n, 1e-12)
        o_ref[...] = jnp.maximum(r, 0.0)
    return pl.pallas_call(
        body,
        out_shape=jax.ShapeDtypeStruct((N, D), jnp.float32),
    )(h, agg, Ws, Wn, b.reshape(1, D))


def _sc_segmax(feat, edge_index):
    mesh = plsc.VectorSubcoreMesh(
        core_axis_name="c", subcore_axis_name="s",
        num_cores=NC, num_subcores=NS)

    @functools.partial(
        pl.kernel,
        out_type=jax.ShapeDtypeStruct((N, D), jnp.float32),
        mesh=mesh,
        compiler_params=pltpu.CompilerParams(needs_layout_passes=False),
        scratch_types=[
            pltpu.VMEM((NPW + 1, D), jnp.float32),
            pltpu.VMEM((WIN,), jnp.int32),
            pltpu.VMEM((WIN,), jnp.int32),
            pltpu.VMEM((WIN,), jnp.int32),
            pltpu.VMEM((WIN,), jnp.int32),
            pltpu.VMEM((64,), jnp.int32),
            pltpu.VMEM((64,), jnp.int32),
            pltpu.VMEM((WIN + 16,), jnp.int32),
            pltpu.VMEM((CHUNK,), jnp.int32),
            pltpu.VMEM((CHUNK,), jnp.int32),
            pltpu.VMEM((2, CHUNK, D), jnp.float32),
            pltpu.VMEM_SHARED((N, D), jnp.float32),
            pltpu.SemaphoreType.DMA,
            pltpu.SemaphoreType.DMA,
            pltpu.SemaphoreType.DMA,
            pltpu.SemaphoreType.DMA,
        ],
    )
    def segmax(feat_hbm, src_hbm, dst_hbm, out_hbm,
               aggl, sbuf0, sbuf1, dbuf0, dbuf1, cbuf, obuf, mbuf,
               idx0, idx1, rows, feat_sh,
               sem_e0, sem_e1, sem_g0, sem_g1):
        wid = lax.axis_index("s") * NC + lax.axis_index("c")
        lo = wid * NPW
        hi = lo + NPW
        sem_e = [sem_e0, sem_e1]
        sem_g = [sem_g0, sem_g1]
        idxb = [idx0, idx1]
        lane = jnp.arange(16, dtype=jnp.int32)
        lane0 = lane == 0

        @pl.when(lax.axis_index("s") == 0)
        def _():
            pltpu.sync_copy(feat_hbm, feat_sh)
        plsc.subcore_barrier()

        @plsc.parallel_loop(0, NPW * D // 16, unroll=8)
        def _(i):
            aggl[i // (D // 16), pl.ds((i % (D // 16)) * 16, 16)] = (
                jnp.zeros((16,), jnp.float32))
        for g in range(4):
            cbuf[pl.ds(g * 16, 16)] = jnp.zeros((16,), jnp.int32)

        sbufs = [sbuf0, sbuf1]
        dbufs = [dbuf0, dbuf1]

        def edge_copies(win, b, s):
            eb = win * WIN
            return (pltpu.make_async_copy(src_hbm.at[pl.ds(eb, WIN)],
                                          sbufs[b], s),
                    pltpu.make_async_copy(dst_hbm.at[pl.ds(eb, WIN)],
                                          dbufs[b], s))

        def gather_copy(b):
            return pltpu.make_async_copy(feat_sh.at[idxb[b]], rows.at[b],
                                         sem_g[b])

        def build_idx(ch, b):
            v = mbuf[pl.ds(ch * CHUNK, CHUNK)]
            idxb[b][...] = jnp.minimum(v & 0x3FFF, N - 1)

        for cp in edge_copies(0, 0, sem_e0):
            cp.start()

        def window(win, b):
            srcv = sbufs[b]
            dstv = dbufs[b]

            @pl.when(win + 1 < NWIN)
            def _():
                for cp in edge_copies(win + 1, 1 - b, sem_e[1 - b]):
                    cp.start()
            for cp in edge_copies(win, b, sem_e[b]):
                cp.wait()

            @plsc.parallel_loop(0, NV, unroll=4)
            def _(v):
                d = dstv[pl.ds(v * 16, 16)]
                m = (d >= lo) & (d < hi)
                cnt = plsc.all_reduce_population_count(m)
                plsc.store_scatter(cbuf, [jnp.full((16,), v, jnp.int32)],
                                   cnt, mask=lane0)

            total = jnp.int32(0)
            for g in range(4):
                c = cbuf[pl.ds(g * 16, 16)]
                ex = plsc.cumsum(c) - c + total
                obuf[pl.ds(g * 16, 16)] = ex
                total = (ex + c)[15]
            nmatch = total

            @plsc.parallel_loop(0, NV, unroll=4)
            def _(v):
                d = dstv[pl.ds(v * 16, 16)]
                s = srcv[pl.ds(v * 16, 16)]
                m = (d >= lo) & (d < hi)
                combo = ((d - lo) << 14) | s
                base = plsc.load_gather(obuf, [jnp.full((16,), v, jnp.int32)])
                pos = base + plsc.cumsum(m.astype(jnp.int32)) - 1
                plsc.store_scatter(mbuf, [pos], combo, mask=m)

            mbuf[pl.ds(nmatch, 16)] = jnp.full((16,), NPW << 14, jnp.int32)

            nch = (nmatch + (CHUNK - 1)) // CHUNK

            @pl.when(nch > 0)
            def _():
                build_idx(0, 0)
                gather_copy(0).start()

            def couter(o, _):
                for cb in range(2):
                    ch = o * 2 + cb

                    @pl.when(ch < nch)
                    def _():
                        gather_copy(cb).wait()

                        @pl.when(ch + 1 < nch)
                        def _():
                            build_idx(ch + 1, 1 - cb)
                            gather_copy(1 - cb).start()

                        rr = rows.at[cb]

                        def rmw(j, _):
                            combo = mbuf[pl.ds(ch * CHUNK + j, 16)][0]
                            dl = combo >> 14
                            for c in range(D // 16):
                                a = aggl[dl, pl.ds(c * 16, 16)]
                                r = rr[j, pl.ds(c * 16, 16)]
                                aggl[dl, pl.ds(c * 16, 16)] = jnp.maximum(a, r)
                            return 0
                        lax.fori_loop(0, CHUNK, rmw, 0, unroll=8)
                return 0
            lax.fori_loop(0, (nch + 1) // 2, couter, 0)
            return 0

        def wouter(o, _):
            for b in range(2):
                window(o * 2 + b, b)
            return 0
        lax.fori_loop(0, NWIN // 2, wouter, 0)

        @pl.when(wid < NW - 1)
        def _():
            pltpu.sync_copy(aggl.at[pl.ds(0, NPW)], out_hbm.at[pl.ds(lo, NPW)])

        @pl.when(wid == NW - 1)
        def _():
            pltpu.sync_copy(aggl.at[pl.ds(0, N - (NW - 1) * NPW)],
                            out_hbm.at[pl.ds((NW - 1) * NPW, N - (NW - 1) * NPW)])

    return segmax(feat, edge_index[0], edge_index[1])


def kernel(inputs, edge_index, Wp1, bp1, Ws1, Wn1, b1, Wp2, bp2, Ws2, Wn2, b2):
    feat1 = _tc_pool(inputs, Wp1, bp1)
    agg1 = _sc_segmax(feat1, edge_index)
    h1 = _tc_out(inputs, agg1, Ws1, Wn1, b1)
    feat2 = _tc_pool(h1, Wp2, bp2)
    agg2 = _sc_segmax(feat2, edge_index)
    h2 = _tc_out(h1, agg2, Ws2, Wn2, b2)
    return h2


# speedup vs baseline: 1.0118x; 1.0032x over previous
import functools

import jax
import jax.numpy as jnp
from jax import lax
from jax.experimental import pallas as pl
from jax.experimental.pallas import tpu as pltpu
from jax.experimental.pallas import tpu_sc as plsc

N = 10000
E = 320000
D = 128

NC = 2
NS = 16
NW = NC * NS
NPW = 320
WIN = 800
NWIN = E // WIN
NV = WIN // 16
CHUNK = 16


def _tc_pool(x, Wp, bp):
    def body(x_ref, w_ref, b_ref, o_ref):
        acc = jnp.dot(x_ref[...], w_ref[...], preferred_element_type=jnp.float32)
        o_ref[...] = jnp.maximum(acc + b_ref[...], 0.0)
    return pl.pallas_call(
        body,
        out_shape=jax.ShapeDtypeStruct((N, D), jnp.float32),
    )(x, Wp, bp.reshape(1, D))


def _tc_out(h, agg, Ws, Wn, b):
    def body(h_ref, a_ref, ws_ref, wn_ref, b_ref, o_ref):
        r = jnp.dot(h_ref[...], ws_ref[...], preferred_element_type=jnp.float32)
        r += jnp.dot(a_ref[...], wn_ref[...], preferred_element_type=jnp.float32)
        r += b_ref[...]
        n = jnp.sqrt(jnp.sum(r * r, axis=1, keepdims=True))
        r = r / jnp.maximum(n, 1e-12)
        o_ref[...] = jnp.maximum(r, 0.0)
    return pl.pallas_call(
        body,
        out_shape=jax.ShapeDtypeStruct((N, D), jnp.float32),
    )(h, agg, Ws, Wn, b.reshape(1, D))


def _sc_segmax(feat, edge_index):
    mesh = plsc.VectorSubcoreMesh(
        core_axis_name="c", subcore_axis_name="s",
        num_cores=NC, num_subcores=NS)

    @functools.partial(
        pl.kernel,
        out_type=jax.ShapeDtypeStruct((N, D), jnp.float32),
        mesh=mesh,
        compiler_params=pltpu.CompilerParams(needs_layout_passes=False),
        scratch_types=[
            pltpu.VMEM((NPW + 1, D), jnp.float32),
            pltpu.VMEM((WIN,), jnp.int32),
            pltpu.VMEM((WIN,), jnp.int32),
            pltpu.VMEM((WIN,), jnp.int32),
            pltpu.VMEM((WIN,), jnp.int32),
            pltpu.VMEM((64,), jnp.int32),
            pltpu.VMEM((64,), jnp.int32),
            pltpu.VMEM((WIN + 16,), jnp.int32),
            pltpu.VMEM((CHUNK,), jnp.int32),
            pltpu.VMEM((CHUNK,), jnp.int32),
            pltpu.VMEM((2, CHUNK, D), jnp.float32),
            pltpu.VMEM_SHARED((N, D), jnp.float32),
            pltpu.SemaphoreType.DMA,
            pltpu.SemaphoreType.DMA,
            pltpu.SemaphoreType.DMA,
            pltpu.SemaphoreType.DMA,
        ],
    )
    def segmax(feat_hbm, src_hbm, dst_hbm, out_hbm,
               aggl, sbuf0, sbuf1, dbuf0, dbuf1, cbuf, obuf, mbuf,
               idx0, idx1, rows, feat_sh,
               sem_e0, sem_e1, sem_g0, sem_g1):
        wid = lax.axis_index("s") * NC + lax.axis_index("c")
        lo = wid * NPW
        hi = lo + NPW
        sem_e = [sem_e0, sem_e1]
        sem_g = [sem_g0, sem_g1]
        idxb = [idx0, idx1]
        lane = jnp.arange(16, dtype=jnp.int32)
        lane0 = lane == 0

        @pl.when(lax.axis_index("s") == 0)
        def _():
            pltpu.sync_copy(feat_hbm, feat_sh)
        plsc.subcore_barrier()

        @plsc.parallel_loop(0, NPW * D // 16, unroll=8)
        def _(i):
            aggl[i // (D // 16), pl.ds((i % (D // 16)) * 16, 16)] = (
                jnp.zeros((16,), jnp.float32))
        for g in range(4):
            cbuf[pl.ds(g * 16, 16)] = jnp.zeros((16,), jnp.int32)

        sbufs = [sbuf0, sbuf1]
        dbufs = [dbuf0, dbuf1]

        def edge_copies(win, b, s):
            eb = win * WIN
            return (pltpu.make_async_copy(src_hbm.at[pl.ds(eb, WIN)],
                                          sbufs[b], s),
                    pltpu.make_async_copy(dst_hbm.at[pl.ds(eb, WIN)],
                                          dbufs[b], s))

        def gather_copy(b):
            return pltpu.make_async_copy(feat_sh.at[idxb[b]], rows.at[b],
                                         sem_g[b])

        def build_idx(ch, b):
            v = mbuf[pl.ds(ch * CHUNK, CHUNK)]
            idxb[b][...] = jnp.minimum(v & 0x3FFF, N - 1)

        for cp in edge_copies(0, 0, sem_e0):
            cp.start()

        def window(win, b):
            srcv = sbufs[b]
            dstv = dbufs[b]

            @pl.when(win + 1 < NWIN)
            def _():
                for cp in edge_copies(win + 1, 1 - b, sem_e[1 - b]):
                    cp.start()
            for cp in edge_copies(win, b, sem_e[b]):
                cp.wait()

            @plsc.parallel_loop(0, NV, unroll=8)
            def _(v):
                d = dstv[pl.ds(v * 16, 16)]
                m = (d >= lo) & (d < hi)
                cnt = plsc.all_reduce_population_count(m)
                plsc.store_scatter(cbuf, [jnp.full((16,), v, jnp.int32)],
                                   cnt, mask=lane0)

            total = jnp.int32(0)
            for g in range(4):
                c = cbuf[pl.ds(g * 16, 16)]
                ex = plsc.cumsum(c) - c + total
                obuf[pl.ds(g * 16, 16)] = ex
                total = (ex + c)[15]
            nmatch = total

            @plsc.parallel_loop(0, NV, unroll=8)
            def _(v):
                d = dstv[pl.ds(v * 16, 16)]
                s = srcv[pl.ds(v * 16, 16)]
                m = (d >= lo) & (d < hi)
                combo = ((d - lo) << 14) | s
                base = plsc.load_gather(obuf, [jnp.full((16,), v, jnp.int32)])
                pos = base + plsc.cumsum(m.astype(jnp.int32)) - 1
                plsc.store_scatter(mbuf, [pos], combo, mask=m)

            mbuf[pl.ds(nmatch, 16)] = jnp.full((16,), NPW << 14, jnp.int32)

            nch = (nmatch + (CHUNK - 1)) // CHUNK

            @pl.when(nch > 0)
            def _():
                build_idx(0, 0)
                gather_copy(0).start()

            def couter(o, _):
                for cb in range(2):
                    ch = o * 2 + cb

                    @pl.when(ch < nch)
                    def _():
                        gather_copy(cb).wait()

                        @pl.when(ch + 1 < nch)
                        def _():
                            build_idx(ch + 1, 1 - cb)
                            gather_copy(1 - cb).start()

                        rr = rows.at[cb]

                        def rmw(j, _):
                            combo = mbuf[pl.ds(ch * CHUNK + j, 16)][0]
                            dl = combo >> 14
                            for c in range(D // 16):
                                a = aggl[dl, pl.ds(c * 16, 16)]
                                r = rr[j, pl.ds(c * 16, 16)]
                                aggl[dl, pl.ds(c * 16, 16)] = jnp.maximum(a, r)
                            return 0
                        lax.fori_loop(0, CHUNK, rmw, 0, unroll=8)
                return 0
            lax.fori_loop(0, (nch + 1) // 2, couter, 0)
            return 0

        def wouter(o, _):
            for b in range(2):
                window(o * 2 + b, b)
            return 0
        lax.fori_loop(0, NWIN // 2, wouter, 0)

        @pl.when(wid < NW - 1)
        def _():
            pltpu.sync_copy(aggl.at[pl.ds(0, NPW)], out_hbm.at[pl.ds(lo, NPW)])

        @pl.when(wid == NW - 1)
        def _():
            pltpu.sync_copy(aggl.at[pl.ds(0, N - (NW - 1) * NPW)],
                            out_hbm.at[pl.ds((NW - 1) * NPW, N - (NW - 1) * NPW)])

    return segmax(feat, edge_index[0], edge_index[1])


def kernel(inputs, edge_index, Wp1, bp1, Ws1, Wn1, b1, Wp2, bp2, Ws2, Wn2, b2):
    feat1 = _tc_pool(inputs, Wp1, bp1)
    agg1 = _sc_segmax(feat1, edge_index)
    h1 = _tc_out(inputs, agg1, Ws1, Wn1, b1)
    feat2 = _tc_pool(h1, Wp2, bp2)
    agg2 = _sc_segmax(feat2, edge_index)
    h2 = _tc_out(h1, agg2, Ws2, Wn2, b2)
    return h2
